# baseline (speedup 1.0000x reference)
import functools

import jax
import jax.numpy as jnp
from jax import lax
from jax.experimental import pallas as pl
from jax.experimental.pallas import tpu as pltpu
from jax.experimental.pallas import tpu_sc as plsc

_VOCAB = 1000000
_D = 64
_B = 1024
_S = 20
_ROWS = 60
_SEG = _B * (1 + _S)
_NC = 2
_NS = 16
_NW = _NC * _NS
_PER_W = _SEG // _NW
_C = 2
_NCHUNK = _PER_W // _C
_NBUF = 2
_NSTEP = _NCHUNK // _NBUF
_NSLICE = _D // 16

_FB = 512
_FMT_GRID = -(-_VOCAB // _FB)
_ROWS128 = _FMT_GRID * (_FB // 2)


def _tc_fmt_body(tabt_ref, out_ref):
    blkt = tabt_ref[...].T
    out_ref[:, 0:_D] = blkt[0 : _FB // 2]
    out_ref[:, _D:128] = blkt[_FB // 2 : _FB]


def _sc_body(idx_hbm, table_hbm, out_hbm, idx_v, rows_v, out_v, gsem0, gsem1):
    gsems = (gsem0, gsem1)
    wid = lax.axis_index("s") * _NC + lax.axis_index("c")
    base = wid * _PER_W

    pltpu.sync_copy(idx_hbm.at[wid], idx_v)

    def gather_start(j, b):
        pltpu.async_copy(table_hbm.at[idx_v.at[j]], rows_v.at[b], gsems[b])

    def gather_wait(b):
        pltpu.make_async_copy(
            table_hbm.at[idx_v.at[0]], rows_v.at[b], gsems[b]
        ).wait()

    for b in range(_NBUF):
        gather_start(b, b)

    def step(i, carry):
        for b in range(_NBUF):
            j = i * _NBUF + b
            gather_wait(b)
            for c in range(_C):
                accs = [
                    rows_v[b, c * _ROWS, pl.ds(16 * k, 16)]
                    for k in range(_NSLICE)
                ]
                for r in range(1, _ROWS):
                    for k in range(_NSLICE):
                        accs[k] = accs[k] + rows_v[
                            b, c * _ROWS + r, pl.ds(16 * k, 16)
                        ]
                seg = j * _C + c
                for k in range(_NSLICE):
                    out_v[seg, pl.ds(16 * k, 16)] = accs[k]
            nj = j + _NBUF

            @pl.when(nj < _NCHUNK)
            def _():
                gather_start(nj, b)

        return carry

    lax.fori_loop(0, _NSTEP, step, 0)
    pltpu.sync_copy(out_v, out_hbm.at[pl.ds(base, _PER_W)])


@jax.jit
def kernel(sub_index, derived_sub_indices, action_mask, table):
    mesh = plsc.VectorSubcoreMesh(core_axis_name="c", subcore_axis_name="s")

    tab128 = pl.pallas_call(
        _tc_fmt_body,
        grid=(_FMT_GRID,),
        in_specs=[pl.BlockSpec((_D, _FB), lambda c: (0, c))],
        out_specs=pl.BlockSpec((_FB // 2, 128), lambda c: (c, 0)),
        out_shape=jax.ShapeDtypeStruct((_ROWS128, 128), jnp.float32),
    )(table.T)
    tab_lin = tab128.reshape(2 * _ROWS128, _D)

    obs_idx = sub_index.reshape(_B, _ROWS).astype(jnp.int32)
    act_idx = derived_sub_indices.reshape(_B * _S, _ROWS).astype(jnp.int32)
    idx = jnp.concatenate([obs_idx, act_idx], axis=0)
    q = idx >> 8
    idx = ((q >> 1) << 9) + ((idx & 255) << 1) + (q & 1)
    idx3 = idx.reshape(_NW, _NCHUNK, _C * _ROWS)

    kfn = functools.partial(
        pl.kernel,
        out_type=jax.ShapeDtypeStruct((_SEG, _D), jnp.float32),
        mesh=mesh,
        compiler_params=pltpu.CompilerParams(use_tc_tiling_on_sc=False),
        scratch_types=[
            pltpu.VMEM((_NCHUNK, _C * _ROWS), jnp.int32),
            pltpu.VMEM((_NBUF, _C * _ROWS, _D), jnp.float32),
            pltpu.VMEM((_PER_W, _D), jnp.float32),
            pltpu.SemaphoreType.DMA,
            pltpu.SemaphoreType.DMA,
        ],
    )(_sc_body)

    out = kfn(idx3, tab_lin)
    obs = out[:_B]
    action = out[_B:].reshape(_B, _S, _D)
    return (obs, action, action_mask)


# speedup vs baseline: 2.2445x; 2.2445x over previous
import functools

import jax
import jax.numpy as jnp
from jax import lax
from jax.experimental import pallas as pl
from jax.experimental.pallas import tpu as pltpu
from jax.experimental.pallas import tpu_sc as plsc

_VOCAB = 1000000
_D = 64
_B = 1024
_S = 20
_ROWS = 60
_SEG = _B * (1 + _S)
_NC = 2
_NS = 16
_NW = _NC * _NS
_PER_W = _SEG // _NW
_C = 2
_NCHUNK = _PER_W // _C
_NBUF = 2
_NSTEP = _NCHUNK // _NBUF
_NSLICE = _D // 16

_FB = 4096
_FMT_GRID = -(-_VOCAB // _FB)
_ROWS128 = _FMT_GRID * (_FB // 2)


def _tc_fmt_body(tabt_ref, out_ref):
    blkt = tabt_ref[...].T
    out_ref[:, 0:_D] = blkt[0 : _FB // 2]
    out_ref[:, _D:128] = blkt[_FB // 2 : _FB]


def _sc_body(idx_hbm, table_hbm, out_hbm, idx_v, rows_v, out_v, gsem0, gsem1):
    gsems = (gsem0, gsem1)
    wid = lax.axis_index("s") * _NC + lax.axis_index("c")
    base = wid * _PER_W

    pltpu.sync_copy(idx_hbm.at[wid], idx_v)

    def gather_start(j, b):
        pltpu.async_copy(table_hbm.at[idx_v.at[j]], rows_v.at[b], gsems[b])

    def gather_wait(b):
        pltpu.make_async_copy(
            table_hbm.at[idx_v.at[0]], rows_v.at[b], gsems[b]
        ).wait()

    for b in range(_NBUF):
        gather_start(b, b)

    def step(i, carry):
        for b in range(_NBUF):
            j = i * _NBUF + b
            gather_wait(b)
            for c in range(_C):
                accs = [
                    rows_v[b, c * _ROWS, pl.ds(16 * k, 16)]
                    for k in range(_NSLICE)
                ]
                for r in range(1, _ROWS):
                    for k in range(_NSLICE):
                        accs[k] = accs[k] + rows_v[
                            b, c * _ROWS + r, pl.ds(16 * k, 16)
                        ]
                seg = j * _C + c
                for k in range(_NSLICE):
                    out_v[seg, pl.ds(16 * k, 16)] = accs[k]
            nj = j + _NBUF

            @pl.when(nj < _NCHUNK)
            def _():
                gather_start(nj, b)

        return carry

    lax.fori_loop(0, _NSTEP, step, 0)
    pltpu.sync_copy(out_v, out_hbm.at[pl.ds(base, _PER_W)])


@jax.jit
def kernel(sub_index, derived_sub_indices, action_mask, table):
    mesh = plsc.VectorSubcoreMesh(core_axis_name="c", subcore_axis_name="s")

    tab128 = pl.pallas_call(
        _tc_fmt_body,
        grid=(_FMT_GRID,),
        in_specs=[pl.BlockSpec((_D, _FB), lambda c: (0, c))],
        out_specs=pl.BlockSpec((_FB // 2, 128), lambda c: (c, 0)),
        out_shape=jax.ShapeDtypeStruct((_ROWS128, 128), jnp.float32),
    )(table.T)
    tab_lin = tab128.reshape(2 * _ROWS128, _D)

    obs_idx = sub_index.reshape(_B, _ROWS).astype(jnp.int32)
    act_idx = derived_sub_indices.reshape(_B * _S, _ROWS).astype(jnp.int32)
    idx = jnp.concatenate([obs_idx, act_idx], axis=0)
    q = idx >> 11
    idx = ((q >> 1) << 12) + ((idx & 2047) << 1) + (q & 1)
    idx3 = idx.reshape(_NW, _NCHUNK, _C * _ROWS)

    kfn = functools.partial(
        pl.kernel,
        out_type=jax.ShapeDtypeStruct((_SEG, _D), jnp.float32),
        mesh=mesh,
        compiler_params=pltpu.CompilerParams(use_tc_tiling_on_sc=False),
        scratch_types=[
            pltpu.VMEM((_NCHUNK, _C * _ROWS), jnp.int32),
            pltpu.VMEM((_NBUF, _C * _ROWS, _D), jnp.float32),
            pltpu.VMEM((_PER_W, _D), jnp.float32),
            pltpu.SemaphoreType.DMA,
            pltpu.SemaphoreType.DMA,
        ],
    )(_sc_body)

    out = kfn(idx3, tab_lin)
    obs = out[:_B]
    action = out[_B:].reshape(_B, _S, _D)
    return (obs, action, action_mask)
